# Initial kernel scaffold; baseline (speedup 1.0000x reference)
import functools

import jax
import jax.numpy as jnp
from jax import lax
from jax.experimental import pallas as pl
from jax.experimental.pallas import tpu as pltpu
from jax.experimental.pallas import tpu_sc as plsc

N = 10000
H = 128
E = 320000
G = 16

NC = 2
NS = 16
NW = NC * NS
EW = E // NW
K = 128
NFULL = EW // K
KT = EW - NFULL * K
RPT = N // NS
WC = 125

BN = 1000
NB = N // BN

_mesh = plsc.VectorSubcoreMesh(
    core_axis_name="c", subcore_axis_name="s", num_cores=NC, num_subcores=NS)


@functools.partial(
    pl.kernel,
    out_type=jax.ShapeDtypeStruct((NW, N), jnp.float32),
    mesh=_mesh,
    scratch_types=[
        pltpu.VMEM((EW,), jnp.int32),
        pltpu.VMEM((N,), jnp.float32),
    ],
)
def _sc_degree(dst_hbm, out_hbm, dstbuf, cnt):
    c = lax.axis_index("c")
    s = lax.axis_index("s")
    wid = s * NC + c
    zero16 = jnp.zeros((16,), jnp.float32)

    @pl.loop(0, N // 16)
    def _zero(i):
        cnt[pl.ds(i * 16, 16)] = zero16

    pltpu.sync_copy(dst_hbm.at[pl.ds(wid * EW, EW)], dstbuf)
    ones16 = jnp.full((16,), 1.0, jnp.float32)

    @pl.loop(0, EW // 16)
    def _count(i):
        idx = dstbuf[pl.ds(i * 16, 16)]
        plsc.addupdate_scatter(cnt, [idx], ones16)

    pltpu.sync_copy(cnt, out_hbm.at[wid])


@functools.partial(
    pl.kernel,
    out_type=jax.ShapeDtypeStruct((NC, N, H), jnp.float32),
    mesh=_mesh,
    scratch_types=[
        pltpu.VMEM((K,), jnp.int32),
        pltpu.VMEM((K,), jnp.int32),
        pltpu.VMEM((K, H), jnp.float32),
        pltpu.VMEM((KT,), jnp.int32),
        pltpu.VMEM((KT,), jnp.int32),
        pltpu.VMEM((KT, H), jnp.float32),
        pltpu.VMEM((K, H), jnp.float32),
        pltpu.VMEM_SHARED((N, H), jnp.float32),
        pltpu.SemaphoreType.DMA,
    ],
)
def _sc_scatter(u_hbm, src_hbm, dst_hbm, out_hbm,
                sidx, didx, rows, sidx2, didx2, rows2, zbuf, acc, sem):
    c = lax.axis_index("c")
    s = lax.axis_index("s")
    wid = s * NC + c
    zero16 = jnp.zeros((16,), jnp.float32)

    @pl.loop(0, K)
    def _zb(i):
        for j in range(H // 16):
            zbuf[i, pl.ds(j * 16, 16)] = zero16

    base = s * RPT

    @pl.loop(0, 4)
    def _zacc(j):
        pltpu.sync_copy(zbuf, acc.at[pl.ds(base + j * K, K)])

    pltpu.sync_copy(zbuf.at[pl.ds(0, RPT - 4 * K)],
                    acc.at[pl.ds(base + 4 * K, RPT - 4 * K)])
    plsc.subcore_barrier()

    ebase = wid * EW

    @pl.loop(0, NFULL)
    def _chunk(i):
        off = ebase + i * K
        pltpu.sync_copy(src_hbm.at[pl.ds(off, K)], sidx)
        pltpu.sync_copy(dst_hbm.at[pl.ds(off, K)], didx)
        pltpu.async_copy(u_hbm.at[sidx], rows, sem).wait()
        pltpu.sync_copy(rows, acc.at[didx], add=True)

    off2 = ebase + NFULL * K
    pltpu.sync_copy(src_hbm.at[pl.ds(off2, KT)], sidx2)
    pltpu.sync_copy(dst_hbm.at[pl.ds(off2, KT)], didx2)
    pltpu.async_copy(u_hbm.at[sidx2], rows2, sem).wait()
    pltpu.sync_copy(rows2, acc.at[didx2], add=True)

    plsc.subcore_barrier()

    @pl.loop(0, RPT // WC)
    def _wout(j):
        r0 = s * RPT + j * WC
        pltpu.sync_copy(acc.at[pl.ds(r0, WC)], out_hbm.at[c, pl.ds(r0, WC)])


def _tc1_body(degp_ref, x_ref, w_ref, u_ref, dis_ref):
    deg = 1.0 + jnp.sum(degp_ref[...], axis=0)
    dis = lax.rsqrt(deg)[:, None]
    u_ref[...] = dis * jnp.dot(x_ref[...], w_ref[...],
                               preferred_element_type=jnp.float32)
    dis_ref[...] = dis


def _tc1(deg_parts, x, w1):
    return pl.pallas_call(
        _tc1_body,
        grid=(NB,),
        in_specs=[
            pl.BlockSpec((NW, BN), lambda i: (0, i)),
            pl.BlockSpec((BN, H), lambda i: (i, 0)),
            pl.BlockSpec((H, H), lambda i: (0, 0)),
        ],
        out_specs=[
            pl.BlockSpec((BN, H), lambda i: (i, 0)),
            pl.BlockSpec((BN, 1), lambda i: (i, 0)),
        ],
        out_shape=[
            jax.ShapeDtypeStruct((N, H), jnp.float32),
            jax.ShapeDtypeStruct((N, 1), jnp.float32),
        ],
    )(deg_parts, x, w1)


def _tc_layer_body(s_ref, u_ref, dis_ref, b_ref, w_ref, out_ref):
    st = s_ref[0] + s_ref[1]
    h = jnp.maximum(dis_ref[...] * (st + u_ref[...]) + b_ref[...], 0.0)
    out_ref[...] = dis_ref[...] * jnp.dot(h, w_ref[...],
                                          preferred_element_type=jnp.float32)


def _tc_layer(s_parts, u, dis, b, w_next):
    return pl.pallas_call(
        _tc_layer_body,
        grid=(NB,),
        in_specs=[
            pl.BlockSpec((NC, BN, H), lambda i: (0, i, 0)),
            pl.BlockSpec((BN, H), lambda i: (i, 0)),
            pl.BlockSpec((BN, 1), lambda i: (i, 0)),
            pl.BlockSpec((1, H), lambda i: (0, 0)),
            pl.BlockSpec((H, H), lambda i: (0, 0)),
        ],
        out_specs=pl.BlockSpec((BN, H), lambda i: (i, 0)),
        out_shape=jax.ShapeDtypeStruct((N, H), jnp.float32),
    )(s_parts, u, dis, b, w_next)


def _tc_head_body(s_ref, u_ref, dis_ref, b_ref, batch_ref,
                  wm1_ref, bm1_ref, wm2_ref, bm2_ref, out_ref, sums, cnts):
    i = pl.program_id(0)

    @pl.when(i == 0)
    def _():
        sums[...] = jnp.zeros_like(sums)
        cnts[...] = jnp.zeros_like(cnts)

    st = s_ref[0] + s_ref[1]
    h = jnp.maximum(dis_ref[...] * (st + u_ref[...]) + b_ref[...], 0.0)
    seg = batch_ref[0, :]
    gids = lax.broadcasted_iota(jnp.int32, (G, BN), 0)
    onehot = (gids == seg[None, :]).astype(jnp.float32)
    sums[...] += jnp.dot(onehot, h, preferred_element_type=jnp.float32)
    cnts[...] += jnp.sum(onehot, axis=1)[:, None]

    @pl.when(i == pl.num_programs(0) - 1)
    def _():
        pooled = sums[...] / jnp.maximum(cnts[...], 1.0)
        z = jnp.maximum(jnp.dot(pooled, wm1_ref[...],
                                preferred_element_type=jnp.float32)
                        + bm1_ref[...], 0.0)
        out_ref[...] = jnp.dot(z, wm2_ref[...],
                               preferred_element_type=jnp.float32) + bm2_ref[...]


def _tc_head(s_parts, u, dis, b, batch2d, wm1, bm1, wm2, bm2):
    return pl.pallas_call(
        _tc_head_body,
        grid=(NB,),
        in_specs=[
            pl.BlockSpec((NC, BN, H), lambda i: (0, i, 0)),
            pl.BlockSpec((BN, H), lambda i: (i, 0)),
            pl.BlockSpec((BN, 1), lambda i: (i, 0)),
            pl.BlockSpec((1, H), lambda i: (0, 0)),
            pl.BlockSpec((1, BN), lambda i: (0, i)),
            pl.BlockSpec((H, H // 2), lambda i: (0, 0)),
            pl.BlockSpec((1, H // 2), lambda i: (0, 0)),
            pl.BlockSpec((H // 2, 1), lambda i: (0, 0)),
            pl.BlockSpec((1, 1), lambda i: (0, 0)),
        ],
        out_specs=pl.BlockSpec((G, 1), lambda i: (0, 0)),
        out_shape=jax.ShapeDtypeStruct((G, 1), jnp.float32),
        scratch_shapes=[
            pltpu.VMEM((G, H), jnp.float32),
            pltpu.VMEM((G, H), jnp.float32),
        ],
    )(s_parts, u, dis, b, batch2d, wm1, bm1, wm2, bm2)


def kernel(x, edge_index, batch, W1, b1, W2, b2, W3, b3, Wm1, bm1, Wm2, bm2):
    src = edge_index[0]
    dst = edge_index[1]

    deg_parts = _sc_degree(dst)
    u1, dis = _tc1(deg_parts, x, W1)

    s1 = _sc_scatter(u1, src, dst)
    u2 = _tc_layer(s1, u1, dis, b1.reshape(1, H), W2)

    s2 = _sc_scatter(u2, src, dst)
    u3 = _tc_layer(s2, u2, dis, b2.reshape(1, H), W3)

    s3 = _sc_scatter(u3, src, dst)
    out = _tc_head(s3, u3, dis, b3.reshape(1, H), batch.reshape(1, N),
                   Wm1, bm1.reshape(1, H // 2), Wm2, bm2.reshape(1, 1))
    return out.reshape(-1)


# speedup vs baseline: 14.2985x; 14.2985x over previous
import functools

import jax
import jax.numpy as jnp
from jax import lax
from jax.experimental import pallas as pl
from jax.experimental.pallas import tpu as pltpu
from jax.experimental.pallas import tpu_sc as plsc

N = 10000
H = 128
E = 320000
G = 16

NC = 2
NS = 16
NW = NC * NS
EW = E // NW
K = 128
NFULL = EW // K
KT = EW - NFULL * K
WC = 80
NCH = N // WC
CPT = -(-NCH // NS)

BN = 1000
NB = N // BN

_mesh = plsc.VectorSubcoreMesh(
    core_axis_name="c", subcore_axis_name="s", num_cores=NC, num_subcores=NS)


@functools.partial(
    pl.kernel,
    out_type=jax.ShapeDtypeStruct((NC, N, 16), jnp.float32),
    mesh=_mesh,
    scratch_types=[
        pltpu.VMEM((K,), jnp.int32),
        pltpu.VMEM((KT,), jnp.int32),
        pltpu.VMEM((K, 16), jnp.float32),
        pltpu.VMEM((WC, 16), jnp.float32),
        pltpu.VMEM_SHARED((N, 16), jnp.float32),
    ],
)
def _sc_degree(dst_hbm, out_hbm, didx, didx2, onesbuf, zbuf, acc):
    c = lax.axis_index("c")
    s = lax.axis_index("s")
    wid = s * NC + c
    zero16 = jnp.zeros((16,), jnp.float32)
    ones16 = jnp.full((16,), 1.0, jnp.float32)

    @pl.loop(0, K)
    def _fill(i):
        onesbuf[i, :] = ones16

    @pl.loop(0, WC)
    def _zb(i):
        zbuf[i, :] = zero16

    @pl.loop(0, CPT)
    def _zacc(j):
        q = s + NS * j

        @pl.when(q < NCH)
        def _():
            pltpu.sync_copy(zbuf, acc.at[pl.ds(q * WC, WC)])

    plsc.subcore_barrier()
    ebase = wid * EW

    @pl.loop(0, NFULL)
    def _chunk(i):
        pltpu.sync_copy(dst_hbm.at[pl.ds(ebase + i * K, K)], didx)
        pltpu.sync_copy(onesbuf, acc.at[didx], add=True)

    pltpu.sync_copy(dst_hbm.at[pl.ds(ebase + NFULL * K, KT)], didx2)
    pltpu.sync_copy(onesbuf.at[pl.ds(0, KT)], acc.at[didx2], add=True)
    plsc.subcore_barrier()

    @pl.loop(0, CPT)
    def _wout(j):
        q = s + NS * j

        @pl.when(q < NCH)
        def _():
            r0 = q * WC
            pltpu.sync_copy(acc.at[pl.ds(r0, WC)],
                            out_hbm.at[c, pl.ds(r0, WC)])


@functools.partial(
    pl.kernel,
    out_type=jax.ShapeDtypeStruct((NC, N, H), jnp.float32),
    mesh=_mesh,
    scratch_types=[
        pltpu.VMEM((K,), jnp.int32),
        pltpu.VMEM((K,), jnp.int32),
        pltpu.VMEM((K, H), jnp.float32),
        pltpu.VMEM((KT,), jnp.int32),
        pltpu.VMEM((KT,), jnp.int32),
        pltpu.VMEM((KT, H), jnp.float32),
        pltpu.VMEM((WC, H), jnp.float32),
        pltpu.VMEM_SHARED((N, H), jnp.float32),
        pltpu.SemaphoreType.DMA,
    ],
)
def _sc_scatter(u_hbm, src_hbm, dst_hbm, out_hbm,
                sidx, didx, rows, sidx2, didx2, rows2, zbuf, acc, sem):
    c = lax.axis_index("c")
    s = lax.axis_index("s")
    wid = s * NC + c
    zero16 = jnp.zeros((16,), jnp.float32)

    @pl.loop(0, WC)
    def _zb(i):
        for j in range(H // 16):
            zbuf[i, pl.ds(j * 16, 16)] = zero16

    @pl.loop(0, CPT)
    def _zacc(j):
        q = s + NS * j

        @pl.when(q < NCH)
        def _():
            pltpu.sync_copy(zbuf, acc.at[pl.ds(q * WC, WC)])

    plsc.subcore_barrier()

    ebase = wid * EW

    @pl.loop(0, NFULL)
    def _chunk(i):
        off = ebase + i * K
        pltpu.sync_copy(src_hbm.at[pl.ds(off, K)], sidx)
        pltpu.sync_copy(dst_hbm.at[pl.ds(off, K)], didx)
        pltpu.async_copy(u_hbm.at[sidx], rows, sem).wait()
        pltpu.sync_copy(rows, acc.at[didx], add=True)

    off2 = ebase + NFULL * K
    pltpu.sync_copy(src_hbm.at[pl.ds(off2, KT)], sidx2)
    pltpu.sync_copy(dst_hbm.at[pl.ds(off2, KT)], didx2)
    pltpu.async_copy(u_hbm.at[sidx2], rows2, sem).wait()
    pltpu.sync_copy(rows2, acc.at[didx2], add=True)

    plsc.subcore_barrier()

    @pl.loop(0, CPT)
    def _wout(j):
        q = s + NS * j

        @pl.when(q < NCH)
        def _():
            r0 = q * WC
            pltpu.sync_copy(acc.at[pl.ds(r0, WC)],
                            out_hbm.at[c, pl.ds(r0, WC)])


def _tc1_body(degp_ref, x_ref, w_ref, u_ref, dis_ref):
    deg = 1.0 + degp_ref[0, :, 0] + degp_ref[1, :, 0]
    dis = lax.rsqrt(deg)[:, None]
    u_ref[...] = dis * jnp.dot(x_ref[...], w_ref[...],
                               preferred_element_type=jnp.float32)
    dis_ref[...] = dis


def _tc1(deg_parts, x, w1):
    return pl.pallas_call(
        _tc1_body,
        grid=(NB,),
        in_specs=[
            pl.BlockSpec((NC, BN, 16), lambda i: (0, i, 0)),
            pl.BlockSpec((BN, H), lambda i: (i, 0)),
            pl.BlockSpec((H, H), lambda i: (0, 0)),
        ],
        out_specs=[
            pl.BlockSpec((BN, H), lambda i: (i, 0)),
            pl.BlockSpec((BN, 1), lambda i: (i, 0)),
        ],
        out_shape=[
            jax.ShapeDtypeStruct((N, H), jnp.float32),
            jax.ShapeDtypeStruct((N, 1), jnp.float32),
        ],
    )(deg_parts, x, w1)


def _tc_layer_body(s_ref, u_ref, dis_ref, b_ref, w_ref, out_ref):
    st = s_ref[0] + s_ref[1]
    h = jnp.maximum(dis_ref[...] * (st + u_ref[...]) + b_ref[...], 0.0)
    out_ref[...] = dis_ref[...] * jnp.dot(h, w_ref[...],
                                          preferred_element_type=jnp.float32)


def _tc_layer(s_parts, u, dis, b, w_next):
    return pl.pallas_call(
        _tc_layer_body,
        grid=(NB,),
        in_specs=[
            pl.BlockSpec((NC, BN, H), lambda i: (0, i, 0)),
            pl.BlockSpec((BN, H), lambda i: (i, 0)),
            pl.BlockSpec((BN, 1), lambda i: (i, 0)),
            pl.BlockSpec((1, H), lambda i: (0, 0)),
            pl.BlockSpec((H, H), lambda i: (0, 0)),
        ],
        out_specs=pl.BlockSpec((BN, H), lambda i: (i, 0)),
        out_shape=jax.ShapeDtypeStruct((N, H), jnp.float32),
    )(s_parts, u, dis, b, w_next)


def _tc_head_body(s_ref, u_ref, dis_ref, b_ref, batch_ref,
                  wm1_ref, bm1_ref, wm2_ref, bm2_ref, out_ref, sums, cnts):
    i = pl.program_id(0)

    @pl.when(i == 0)
    def _():
        sums[...] = jnp.zeros_like(sums)
        cnts[...] = jnp.zeros_like(cnts)

    st = s_ref[0] + s_ref[1]
    h = jnp.maximum(dis_ref[...] * (st + u_ref[...]) + b_ref[...], 0.0)
    seg = batch_ref[0, 0, :]
    gids = lax.broadcasted_iota(jnp.int32, (G, BN), 0)
    onehot = (gids == seg[None, :]).astype(jnp.float32)
    sums[...] += jnp.dot(onehot, h, preferred_element_type=jnp.float32)
    cnts[...] += jnp.sum(onehot, axis=1)[:, None]

    @pl.when(i == pl.num_programs(0) - 1)
    def _():
        pooled = sums[...] / jnp.maximum(cnts[...], 1.0)
        z = jnp.maximum(jnp.dot(pooled, wm1_ref[...],
                                preferred_element_type=jnp.float32)
                        + bm1_ref[...], 0.0)
        out_ref[...] = jnp.dot(z, wm2_ref[...],
                               preferred_element_type=jnp.float32) + bm2_ref[...]


def _tc_head(s_parts, u, dis, b, batch2d, wm1, bm1, wm2, bm2):
    return pl.pallas_call(
        _tc_head_body,
        grid=(NB,),
        in_specs=[
            pl.BlockSpec((NC, BN, H), lambda i: (0, i, 0)),
            pl.BlockSpec((BN, H), lambda i: (i, 0)),
            pl.BlockSpec((BN, 1), lambda i: (i, 0)),
            pl.BlockSpec((1, H), lambda i: (0, 0)),
            pl.BlockSpec((1, 1, BN), lambda i: (i, 0, 0)),
            pl.BlockSpec((H, H // 2), lambda i: (0, 0)),
            pl.BlockSpec((1, H // 2), lambda i: (0, 0)),
            pl.BlockSpec((H // 2, 1), lambda i: (0, 0)),
            pl.BlockSpec((1, 1), lambda i: (0, 0)),
        ],
        out_specs=pl.BlockSpec((G, 1), lambda i: (0, 0)),
        out_shape=jax.ShapeDtypeStruct((G, 1), jnp.float32),
        scratch_shapes=[
            pltpu.VMEM((G, H), jnp.float32),
            pltpu.VMEM((G, H), jnp.float32),
        ],
    )(s_parts, u, dis, b, batch2d, wm1, bm1, wm2, bm2)


def kernel(x, edge_index, batch, W1, b1, W2, b2, W3, b3, Wm1, bm1, Wm2, bm2):
    src = edge_index[0]
    dst = edge_index[1]

    deg_parts = _sc_degree(dst)
    u1, dis = _tc1(deg_parts, x, W1)

    s1 = _sc_scatter(u1, src, dst)
    u2 = _tc_layer(s1, u1, dis, b1.reshape(1, H), W2)

    s2 = _sc_scatter(u2, src, dst)
    u3 = _tc_layer(s2, u2, dis, b2.reshape(1, H), W3)

    s3 = _sc_scatter(u3, src, dst)
    out = _tc_head(s3, u3, dis, b3.reshape(1, H), batch.reshape(NB, 1, BN),
                   Wm1, bm1.reshape(1, H // 2), Wm2, bm2.reshape(1, 1))
    return out.reshape(-1)


# speedup vs baseline: 31.0277x; 2.1700x over previous
import functools

import jax
import jax.numpy as jnp
from jax import lax
from jax.experimental import pallas as pl
from jax.experimental.pallas import tpu as pltpu
from jax.experimental.pallas import tpu_sc as plsc

N = 10000
H = 128
E = 320000
G = 16

NC = 2
NS = 16
NW = NC * NS
EW = E // NW
K = 80
WC = 80
NCH = N // WC
CPT = -(-NCH // NS)

CH2 = EW // K
RING = 4

BN = 1000
NB = N // BN

_mesh = plsc.VectorSubcoreMesh(
    core_axis_name="c", subcore_axis_name="s", num_cores=NC, num_subcores=NS)


@functools.partial(
    pl.kernel,
    out_type=jax.ShapeDtypeStruct((NC, N, 16), jnp.float32),
    mesh=_mesh,
    scratch_types=(
        [pltpu.VMEM((K,), jnp.int32)] * RING
        + [
            pltpu.VMEM((K, 16), jnp.float32),
            pltpu.VMEM((WC, 16), jnp.float32),
            pltpu.VMEM_SHARED((N, 16), jnp.float32),
        ]
        + [pltpu.SemaphoreType.DMA] * (2 * RING)
    ),
)
def _sc_degree(dst_hbm, out_hbm, di0, di1, di2, di3, onesbuf, zbuf, acc,
               is0, is1, is2, is3, ss0, ss1, ss2, ss3):
    didx = [di0, di1, di2, di3]
    isem = [is0, is1, is2, is3]
    ssem = [ss0, ss1, ss2, ss3]
    c = lax.axis_index("c")
    s = lax.axis_index("s")
    wid = s * NC + c
    zero16 = jnp.zeros((16,), jnp.float32)
    ones16 = jnp.full((16,), 1.0, jnp.float32)

    @pl.loop(0, K)
    def _fill(i):
        onesbuf[i, :] = ones16

    @pl.loop(0, WC)
    def _zb(i):
        zbuf[i, :] = zero16

    @pl.loop(0, CPT)
    def _zacc(j):
        q = s + NS * j

        @pl.when(q < NCH)
        def _():
            pltpu.sync_copy(zbuf, acc.at[pl.ds(q * WC, WC)])

    plsc.subcore_barrier()
    ebase = wid * EW
    n_outer = (CH2 + 2 * RING - 1) // RING

    @pl.loop(0, n_outer)
    def _pipe(io):
        for b in range(RING):
            i = io * RING + b

            @pl.when(jnp.logical_and(i >= RING, i - RING < CH2))
            def _(b=b):
                pltpu.make_async_copy(
                    onesbuf, acc.at[didx[b]], ssem[b]).wait()

            @pl.when(i < CH2)
            def _(i=i, b=b):
                pltpu.async_copy(dst_hbm.at[pl.ds(ebase + i * K, K)],
                                 didx[b], isem[b])

            g = i - 1
            bg = (b - 1) % RING

            @pl.when(jnp.logical_and(g >= 0, g < CH2))
            def _(g=g, bg=bg):
                pltpu.make_async_copy(
                    dst_hbm.at[pl.ds(ebase + g * K, K)], didx[bg],
                    isem[bg]).wait()
                pltpu.async_copy(onesbuf, acc.at[didx[bg]], ssem[bg],
                                 add=True)

    plsc.subcore_barrier()

    @pl.loop(0, CPT)
    def _wout(j):
        q = s + NS * j

        @pl.when(q < NCH)
        def _():
            r0 = q * WC
            pltpu.sync_copy(acc.at[pl.ds(r0, WC)],
                            out_hbm.at[c, pl.ds(r0, WC)])


@functools.partial(
    pl.kernel,
    out_type=jax.ShapeDtypeStruct((NC, N, H), jnp.float32),
    mesh=_mesh,
    scratch_types=(
        [pltpu.VMEM((K,), jnp.int32)] * (2 * RING)
        + [pltpu.VMEM((K, H), jnp.float32)] * RING
        + [pltpu.VMEM_SHARED((N, H), jnp.float32)]
        + [pltpu.SemaphoreType.DMA] * (3 * RING)
    ),
)
def _sc_scatter(u_hbm, src_hbm, dst_hbm, out_hbm,
                si0, si1, si2, si3, di0, di1, di2, di3,
                rb0, rb1, rb2, rb3, acc,
                is0, is1, is2, is3,
                gs0, gs1, gs2, gs3,
                ss0, ss1, ss2, ss3):
    sidx = [si0, si1, si2, si3]
    didx = [di0, di1, di2, di3]
    rows = [rb0, rb1, rb2, rb3]
    isem = [is0, is1, is2, is3]
    gsem = [gs0, gs1, gs2, gs3]
    ssem = [ss0, ss1, ss2, ss3]
    c = lax.axis_index("c")
    s = lax.axis_index("s")
    wid = s * NC + c
    zero16 = jnp.zeros((16,), jnp.float32)

    @pl.loop(0, WC)
    def _zb(i):
        for j in range(H // 16):
            rb0[i, pl.ds(j * 16, 16)] = zero16

    @pl.loop(0, CPT)
    def _zacc(j):
        q = s + NS * j

        @pl.when(q < NCH)
        def _():
            pltpu.sync_copy(rb0.at[pl.ds(0, WC)], acc.at[pl.ds(q * WC, WC)])

    plsc.subcore_barrier()

    ebase = wid * EW
    n_outer = (CH2 + 2 * RING - 1) // RING

    @pl.loop(0, n_outer)
    def _pipe(io):
        for b in range(RING):
            i = io * RING + b

            @pl.when(jnp.logical_and(i >= RING, i - RING < CH2))
            def _(b=b):
                pltpu.make_async_copy(
                    rows[b], acc.at[didx[b]], ssem[b]).wait()

            @pl.when(i < CH2)
            def _(i=i, b=b):
                off = ebase + i * K
                pltpu.async_copy(src_hbm.at[pl.ds(off, K)], sidx[b], isem[b])
                pltpu.async_copy(dst_hbm.at[pl.ds(off, K)], didx[b], isem[b])

            g = i - 1
            bg = (b - 1) % RING

            @pl.when(jnp.logical_and(g >= 0, g < CH2))
            def _(g=g, bg=bg):
                off = ebase + g * K
                pltpu.make_async_copy(
                    src_hbm.at[pl.ds(off, K)], sidx[bg], isem[bg]).wait()
                pltpu.make_async_copy(
                    dst_hbm.at[pl.ds(off, K)], didx[bg], isem[bg]).wait()
                pltpu.async_copy(u_hbm.at[sidx[bg]], rows[bg], gsem[bg])

            j = i - 3
            bj = (b - 3) % RING

            @pl.when(jnp.logical_and(j >= 0, j < CH2))
            def _(bj=bj):
                pltpu.make_async_copy(
                    u_hbm.at[sidx[bj]], rows[bj], gsem[bj]).wait()
                pltpu.async_copy(rows[bj], acc.at[didx[bj]], ssem[bj],
                                 add=True)

    plsc.subcore_barrier()

    @pl.loop(0, CPT)
    def _wout(j):
        q = s + NS * j

        @pl.when(q < NCH)
        def _():
            r0 = q * WC
            pltpu.sync_copy(acc.at[pl.ds(r0, WC)],
                            out_hbm.at[c, pl.ds(r0, WC)])


def _tc1_body(degp_ref, x_ref, w_ref, u_ref, dis_ref):
    deg = 1.0 + degp_ref[0, :, 0] + degp_ref[1, :, 0]
    dis = lax.rsqrt(deg)[:, None]
    u_ref[...] = dis * jnp.dot(x_ref[...], w_ref[...],
                               preferred_element_type=jnp.float32)
    dis_ref[...] = dis


def _tc1(deg_parts, x, w1):
    return pl.pallas_call(
        _tc1_body,
        grid=(NB,),
        in_specs=[
            pl.BlockSpec((NC, BN, 16), lambda i: (0, i, 0)),
            pl.BlockSpec((BN, H), lambda i: (i, 0)),
            pl.BlockSpec((H, H), lambda i: (0, 0)),
        ],
        out_specs=[
            pl.BlockSpec((BN, H), lambda i: (i, 0)),
            pl.BlockSpec((BN, 1), lambda i: (i, 0)),
        ],
        out_shape=[
            jax.ShapeDtypeStruct((N, H), jnp.float32),
            jax.ShapeDtypeStruct((N, 1), jnp.float32),
        ],
    )(deg_parts, x, w1)


def _tc_layer_body(s_ref, u_ref, dis_ref, b_ref, w_ref, out_ref):
    st = s_ref[0] + s_ref[1]
    h = jnp.maximum(dis_ref[...] * (st + u_ref[...]) + b_ref[...], 0.0)
    out_ref[...] = dis_ref[...] * jnp.dot(h, w_ref[...],
                                          preferred_element_type=jnp.float32)


def _tc_layer(s_parts, u, dis, b, w_next):
    return pl.pallas_call(
        _tc_layer_body,
        grid=(NB,),
        in_specs=[
            pl.BlockSpec((NC, BN, H), lambda i: (0, i, 0)),
            pl.BlockSpec((BN, H), lambda i: (i, 0)),
            pl.BlockSpec((BN, 1), lambda i: (i, 0)),
            pl.BlockSpec((1, H), lambda i: (0, 0)),
            pl.BlockSpec((H, H), lambda i: (0, 0)),
        ],
        out_specs=pl.BlockSpec((BN, H), lambda i: (i, 0)),
        out_shape=jax.ShapeDtypeStruct((N, H), jnp.float32),
    )(s_parts, u, dis, b, w_next)


def _tc_head_body(s_ref, u_ref, dis_ref, b_ref, batch_ref,
                  wm1_ref, bm1_ref, wm2_ref, bm2_ref, out_ref, sums, cnts):
    i = pl.program_id(0)

    @pl.when(i == 0)
    def _():
        sums[...] = jnp.zeros_like(sums)
        cnts[...] = jnp.zeros_like(cnts)

    st = s_ref[0] + s_ref[1]
    h = jnp.maximum(dis_ref[...] * (st + u_ref[...]) + b_ref[...], 0.0)
    seg = batch_ref[0, 0, :]
    gids = lax.broadcasted_iota(jnp.int32, (G, BN), 0)
    onehot = (gids == seg[None, :]).astype(jnp.float32)
    sums[...] += jnp.dot(onehot, h, preferred_element_type=jnp.float32)
    cnts[...] += jnp.sum(onehot, axis=1)[:, None]

    @pl.when(i == pl.num_programs(0) - 1)
    def _():
        pooled = sums[...] / jnp.maximum(cnts[...], 1.0)
        z = jnp.maximum(jnp.dot(pooled, wm1_ref[...],
                                preferred_element_type=jnp.float32)
                        + bm1_ref[...], 0.0)
        out_ref[...] = jnp.dot(z, wm2_ref[...],
                               preferred_element_type=jnp.float32) + bm2_ref[...]


def _tc_head(s_parts, u, dis, b, batch2d, wm1, bm1, wm2, bm2):
    return pl.pallas_call(
        _tc_head_body,
        grid=(NB,),
        in_specs=[
            pl.BlockSpec((NC, BN, H), lambda i: (0, i, 0)),
            pl.BlockSpec((BN, H), lambda i: (i, 0)),
            pl.BlockSpec((BN, 1), lambda i: (i, 0)),
            pl.BlockSpec((1, H), lambda i: (0, 0)),
            pl.BlockSpec((1, 1, BN), lambda i: (i, 0, 0)),
            pl.BlockSpec((H, H // 2), lambda i: (0, 0)),
            pl.BlockSpec((1, H // 2), lambda i: (0, 0)),
            pl.BlockSpec((H // 2, 1), lambda i: (0, 0)),
            pl.BlockSpec((1, 1), lambda i: (0, 0)),
        ],
        out_specs=pl.BlockSpec((G, 1), lambda i: (0, 0)),
        out_shape=jax.ShapeDtypeStruct((G, 1), jnp.float32),
        scratch_shapes=[
            pltpu.VMEM((G, H), jnp.float32),
            pltpu.VMEM((G, H), jnp.float32),
        ],
    )(s_parts, u, dis, b, batch2d, wm1, bm1, wm2, bm2)


def kernel(x, edge_index, batch, W1, b1, W2, b2, W3, b3, Wm1, bm1, Wm2, bm2):
    src = edge_index[0]
    dst = edge_index[1]

    deg_parts = _sc_degree(dst)
    u1, dis = _tc1(deg_parts, x, W1)

    s1 = _sc_scatter(u1, src, dst)
    u2 = _tc_layer(s1, u1, dis, b1.reshape(1, H), W2)

    s2 = _sc_scatter(u2, src, dst)
    u3 = _tc_layer(s2, u2, dis, b2.reshape(1, H), W3)

    s3 = _sc_scatter(u3, src, dst)
    out = _tc_head(s3, u3, dis, b3.reshape(1, H), batch.reshape(NB, 1, BN),
                   Wm1, bm1.reshape(1, H // 2), Wm2, bm2.reshape(1, 1))
    return out.reshape(-1)
